# Initial kernel scaffold; baseline (speedup 1.0000x reference)
import jax
import jax.numpy as jnp
from jax.experimental import pallas as pl


def kernel(x, edge_index, etypes, weight, w_comp, loop_weight, h_bias, W1, b1, W2, b2):
    raise NotImplementedError("write your pallas kernel here")


# speedup vs baseline: 17.1446x; 17.1446x over previous
import functools

import jax
import jax.numpy as jnp
from jax import lax
from jax.experimental import pallas as pl
from jax.experimental.pallas import tpu as pltpu
from jax.experimental.pallas import tpu_sc as plsc

N_NODES = 10000
N_EDGES = 320000
FEAT = 128
HFEAT = FEAT // 2
NUM_RELS = 8
NUM_BASES = 4

NC = 2
NS = 16
NW = NC * NS

EDGE_CHUNK = 80
CHUNK_ROWS = N_EDGES // EDGE_CHUNK
TILE_CHUNKS = CHUNK_ROWS // NW
ROWS_MAIN = 624
ZROWS = 48

ROW_BLOCK = 1000


def _proj_body(x_ref, w_ref, wc_ref, lw_ref, hall_ref, xl_ref):
    xb = x_ref[...]
    for r in range(NUM_RELS):
        wr = wc_ref[r, 0] * w_ref[0]
        for b in range(1, NUM_BASES):
            wr = wr + wc_ref[r, b] * w_ref[b]
        hall_ref[:, r * FEAT:(r + 1) * FEAT] = jnp.dot(
            xb, wr, preferred_element_type=jnp.float32)
    xl_ref[...] = jnp.dot(xb, lw_ref[...], preferred_element_type=jnp.float32)


def _project(x, weight, w_comp, loop_weight):
    return pl.pallas_call(
        _proj_body,
        grid=(N_NODES // ROW_BLOCK,),
        in_specs=[
            pl.BlockSpec((ROW_BLOCK, FEAT), lambda i: (i, 0)),
            pl.BlockSpec((NUM_BASES, FEAT, FEAT), lambda i: (0, 0, 0)),
            pl.BlockSpec(memory_space=pltpu.SMEM),
            pl.BlockSpec((FEAT, FEAT), lambda i: (0, 0)),
        ],
        out_specs=[
            pl.BlockSpec((ROW_BLOCK, NUM_RELS * FEAT), lambda i: (i, 0)),
            pl.BlockSpec((ROW_BLOCK, FEAT), lambda i: (i, 0)),
        ],
        out_shape=[
            jax.ShapeDtypeStruct((N_NODES, NUM_RELS * FEAT), jnp.float32),
            jax.ShapeDtypeStruct((N_NODES, FEAT), jnp.float32),
        ],
    )(x, weight, w_comp, loop_weight)


def _edge_agg(table, src3, et3, dst3):
    mesh = plsc.VectorSubcoreMesh(core_axis_name="c", subcore_axis_name="s")

    @functools.partial(
        pl.kernel,
        mesh=mesh,
        compiler_params=pltpu.CompilerParams(use_tc_tiling_on_sc=False),
        out_type=jax.ShapeDtypeStruct((2 * NC * N_NODES, HFEAT), jnp.float32),
        scratch_types=[
            pltpu.VMEM((TILE_CHUNKS, EDGE_CHUNK), jnp.int32),
            pltpu.VMEM((TILE_CHUNKS, EDGE_CHUNK), jnp.int32),
            pltpu.VMEM((TILE_CHUNKS, EDGE_CHUNK), jnp.int32),
            pltpu.VMEM((EDGE_CHUNK, HFEAT), jnp.float32),
            pltpu.VMEM((ZROWS, HFEAT), jnp.float32),
            pltpu.VMEM_SHARED((N_NODES, HFEAT), jnp.float32),
            pltpu.SemaphoreType.DMA,
        ],
    )
    def k(table_hbm, src_hbm, et_hbm, dst_hbm, out_hbm,
          src_v, et_v, dst_v, rows_v, zbuf_v, acc_sh, sem):
        cid = lax.axis_index("c")
        tid = lax.axis_index("s")
        wid = tid * NC + cid
        nbase = tid * ROWS_MAIN
        last = tid == NS - 1

        zv = jnp.zeros((16,), jnp.float32)

        def zrow(r, carry):
            for j in range(HFEAT // 16):
                zbuf_v[r, pl.ds(j * 16, 16)] = zv
            return carry

        lax.fori_loop(0, ZROWS, zrow, 0)

        def zero_acc():
            for z in range(ROWS_MAIN // ZROWS):
                pltpu.sync_copy(zbuf_v,
                                acc_sh.at[pl.ds(nbase + z * ZROWS, ZROWS)])

            @pl.when(last)
            def _():
                pltpu.sync_copy(zbuf_v.at[pl.ds(0, 16)],
                                acc_sh.at[pl.ds(N_NODES - 16, 16)])

        def writeback(plane):
            obase = plane * N_NODES + nbase
            pltpu.sync_copy(acc_sh.at[pl.ds(nbase, ROWS_MAIN)],
                            out_hbm.at[pl.ds(obase, ROWS_MAIN)])

            @pl.when(last)
            def _():
                pltpu.sync_copy(
                    acc_sh.at[pl.ds(N_NODES - 16, 16)],
                    out_hbm.at[pl.ds(plane * N_NODES + N_NODES - 16, 16)])

        zero_acc()

        pltpu.sync_copy(src_hbm.at[wid], src_v)
        pltpu.sync_copy(et_hbm.at[wid], et_v)
        pltpu.sync_copy(dst_hbm.at[wid], dst_v)

        def gfill(r, carry):
            for j in range(EDGE_CHUNK // 16):
                s = src_v[r, pl.ds(j * 16, 16)]
                e = et_v[r, pl.ds(j * 16, 16)]
                src_v[r, pl.ds(j * 16, 16)] = (s * NUM_RELS + e) * 2
            return carry

        lax.fori_loop(0, TILE_CHUNKS, gfill, 0)
        plsc.subcore_barrier()

        def chunk(c, carry):
            pltpu.async_copy(table_hbm.at[src_v.at[c]], rows_v, sem).wait()
            pltpu.sync_copy(rows_v, acc_sh.at[dst_v.at[c]], add=True)
            return carry

        lax.fori_loop(0, TILE_CHUNKS, chunk, 0)
        plsc.subcore_barrier()
        writeback(cid)

        def bump(r, carry):
            for j in range(EDGE_CHUNK // 16):
                src_v[r, pl.ds(j * 16, 16)] = src_v[r, pl.ds(j * 16, 16)] + 1
            return carry

        lax.fori_loop(0, TILE_CHUNKS, bump, 0)
        zero_acc()
        plsc.subcore_barrier()
        lax.fori_loop(0, TILE_CHUNKS, chunk, 0)
        plsc.subcore_barrier()
        writeback(NC + cid)

    return k(table, src3, et3, dst3)


def _mlp_body(a00_ref, a01_ref, a10_ref, a11_ref, xl_ref, hb_ref,
              w1_ref, b1_ref, w2_ref, b2_ref, o_ref):
    xl = xl_ref[...]
    hb = hb_ref[...]
    w1 = w1_ref[...]
    s0 = a00_ref[...] + a01_ref[...] + xl[:, :HFEAT] + hb[:, :HFEAT]
    s1 = a10_ref[...] + a11_ref[...] + xl[:, HFEAT:] + hb[:, HFEAT:]
    t = (jnp.dot(s0, w1[:HFEAT, :], preferred_element_type=jnp.float32)
         + jnp.dot(s1, w1[HFEAT:, :], preferred_element_type=jnp.float32)
         + b1_ref[...])
    h1 = jnp.maximum(t, 0.0)
    o_ref[...] = jnp.maximum(
        jnp.dot(h1, w2_ref[...], preferred_element_type=jnp.float32)
        + b2_ref[...], 0.0)


def _mlp(aggp, xl, hb, W1, b1, W2, b2):
    nb = N_NODES // ROW_BLOCK

    def plane_spec(p):
        return pl.BlockSpec((ROW_BLOCK, HFEAT), lambda i, p=p: (i + p * nb, 0))

    row_spec = pl.BlockSpec((ROW_BLOCK, FEAT), lambda i: (i, 0))
    mat_spec = pl.BlockSpec((FEAT, FEAT), lambda i: (0, 0))
    vec_spec = pl.BlockSpec((1, FEAT), lambda i: (0, 0))
    return pl.pallas_call(
        _mlp_body,
        grid=(nb,),
        in_specs=[plane_spec(0), plane_spec(1), plane_spec(2), plane_spec(3),
                  row_spec, vec_spec, mat_spec, vec_spec, mat_spec, vec_spec],
        out_specs=row_spec,
        out_shape=jax.ShapeDtypeStruct((N_NODES, FEAT), jnp.float32),
    )(aggp, aggp, aggp, aggp, xl, hb, W1, b1, W2, b2)


def kernel(x, edge_index, etypes, weight, w_comp, loop_weight, h_bias,
           W1, b1, W2, b2):
    eshape = (NW, TILE_CHUNKS, EDGE_CHUNK)
    src3 = edge_index[0].astype(jnp.int32).reshape(eshape)
    dst3 = edge_index[1].astype(jnp.int32).reshape(eshape)
    et3 = etypes.astype(jnp.int32).reshape(eshape)
    hall, xl = _project(x, weight, w_comp, loop_weight)
    table = hall.reshape(N_NODES * NUM_RELS * 2, HFEAT)
    aggp = _edge_agg(table, src3, et3, dst3)
    return _mlp(aggp, xl, h_bias.reshape(1, FEAT), W1,
                b1.reshape(1, FEAT), W2, b2.reshape(1, FEAT))


# speedup vs baseline: 25.4522x; 1.4846x over previous
import functools

import jax
import jax.numpy as jnp
from jax import lax
from jax.experimental import pallas as pl
from jax.experimental.pallas import tpu as pltpu
from jax.experimental.pallas import tpu_sc as plsc

N_NODES = 10000
N_EDGES = 320000
FEAT = 128
HFEAT = FEAT // 2
NUM_RELS = 8
NUM_BASES = 4

NC = 2
NS = 16
NW = NC * NS

EDGE_CHUNK = 80
CHUNK_ROWS = N_EDGES // EDGE_CHUNK
TILE_CHUNKS = CHUNK_ROWS // NW
ROWS_MAIN = 624
ZROWS = 48

ROW_BLOCK = 1000


def _proj_body(x_ref, w_ref, wc_ref, lw_ref, hall_ref, xl_ref):
    xb = x_ref[...]
    for r in range(NUM_RELS):
        wr = wc_ref[r, 0] * w_ref[0]
        for b in range(1, NUM_BASES):
            wr = wr + wc_ref[r, b] * w_ref[b]
        hall_ref[:, r * FEAT:(r + 1) * FEAT] = jnp.dot(
            xb, wr, preferred_element_type=jnp.float32)
    xl_ref[...] = jnp.dot(xb, lw_ref[...], preferred_element_type=jnp.float32)


def _project(x, weight, w_comp, loop_weight):
    return pl.pallas_call(
        _proj_body,
        grid=(N_NODES // ROW_BLOCK,),
        in_specs=[
            pl.BlockSpec((ROW_BLOCK, FEAT), lambda i: (i, 0)),
            pl.BlockSpec((NUM_BASES, FEAT, FEAT), lambda i: (0, 0, 0)),
            pl.BlockSpec(memory_space=pltpu.SMEM),
            pl.BlockSpec((FEAT, FEAT), lambda i: (0, 0)),
        ],
        out_specs=[
            pl.BlockSpec((ROW_BLOCK, NUM_RELS * FEAT), lambda i: (i, 0)),
            pl.BlockSpec((ROW_BLOCK, FEAT), lambda i: (i, 0)),
        ],
        out_shape=[
            jax.ShapeDtypeStruct((N_NODES, NUM_RELS * FEAT), jnp.float32),
            jax.ShapeDtypeStruct((N_NODES, FEAT), jnp.float32),
        ],
    )(x, weight, w_comp, loop_weight)


def _edge_agg(table, src3, et3, dst3):
    mesh = plsc.VectorSubcoreMesh(core_axis_name="c", subcore_axis_name="s")

    @functools.partial(
        pl.kernel,
        mesh=mesh,
        compiler_params=pltpu.CompilerParams(use_tc_tiling_on_sc=False),
        out_type=jax.ShapeDtypeStruct((2 * NC * N_NODES, HFEAT), jnp.float32),
        scratch_types=[
            pltpu.VMEM((TILE_CHUNKS, EDGE_CHUNK), jnp.int32),
            pltpu.VMEM((TILE_CHUNKS, EDGE_CHUNK), jnp.int32),
            pltpu.VMEM((TILE_CHUNKS, EDGE_CHUNK), jnp.int32),
            pltpu.VMEM((EDGE_CHUNK, HFEAT), jnp.float32),
            pltpu.VMEM((EDGE_CHUNK, HFEAT), jnp.float32),
            pltpu.VMEM((ZROWS, HFEAT), jnp.float32),
            pltpu.VMEM_SHARED((N_NODES, HFEAT), jnp.float32),
            pltpu.SemaphoreType.DMA,
            pltpu.SemaphoreType.DMA,
        ],
    )
    def k(table_hbm, src_hbm, et_hbm, dst_hbm, out_hbm,
          src_v, et_v, dst_v, buf0, buf1, zbuf_v, acc_sh, sem0, sem1):
        cid = lax.axis_index("c")
        tid = lax.axis_index("s")
        wid = tid * NC + cid
        nbase = tid * ROWS_MAIN
        last = tid == NS - 1

        zv = jnp.zeros((16,), jnp.float32)

        def zrow(r, carry):
            for j in range(HFEAT // 16):
                zbuf_v[r, pl.ds(j * 16, 16)] = zv
            return carry

        lax.fori_loop(0, ZROWS, zrow, 0)

        def zero_acc():
            for z in range(ROWS_MAIN // ZROWS):
                pltpu.sync_copy(zbuf_v,
                                acc_sh.at[pl.ds(nbase + z * ZROWS, ZROWS)])

            @pl.when(last)
            def _():
                pltpu.sync_copy(zbuf_v.at[pl.ds(0, 16)],
                                acc_sh.at[pl.ds(N_NODES - 16, 16)])

        def writeback(plane):
            obase = plane * N_NODES + nbase
            pltpu.sync_copy(acc_sh.at[pl.ds(nbase, ROWS_MAIN)],
                            out_hbm.at[pl.ds(obase, ROWS_MAIN)])

            @pl.when(last)
            def _():
                pltpu.sync_copy(
                    acc_sh.at[pl.ds(N_NODES - 16, 16)],
                    out_hbm.at[pl.ds(plane * N_NODES + N_NODES - 16, 16)])

        zero_acc()

        pltpu.sync_copy(src_hbm.at[wid], src_v)
        pltpu.sync_copy(et_hbm.at[wid], et_v)
        pltpu.sync_copy(dst_hbm.at[wid], dst_v)

        def gfill(r, carry):
            for j in range(EDGE_CHUNK // 16):
                s = src_v[r, pl.ds(j * 16, 16)]
                e = et_v[r, pl.ds(j * 16, 16)]
                src_v[r, pl.ds(j * 16, 16)] = (s * NUM_RELS + e) * 2
            return carry

        lax.fori_loop(0, TILE_CHUNKS, gfill, 0)
        plsc.subcore_barrier()

        def gather(c, buf, sem):
            pltpu.async_copy(table_hbm.at[src_v.at[c]], buf, sem)

        def gwait(c, buf, sem):
            pltpu.make_async_copy(table_hbm.at[src_v.at[c]], buf, sem).wait()

        def scatter(c, buf):
            pltpu.sync_copy(buf, acc_sh.at[dst_v.at[c]], add=True)

        def edge_sweep():
            gather(0, buf0, sem0)

            def pair(i, carry):
                a = 2 * i
                gather(a + 1, buf1, sem1)
                gwait(a, buf0, sem0)
                scatter(a, buf0)
                gather(a + 2, buf0, sem0)
                gwait(a + 1, buf1, sem1)
                scatter(a + 1, buf1)
                return carry

            lax.fori_loop(0, (TILE_CHUNKS - 1) // 2, pair, 0)
            last = TILE_CHUNKS - 1
            gwait(last, buf0, sem0)
            scatter(last, buf0)

        edge_sweep()
        plsc.subcore_barrier()
        writeback(cid)

        def bump(r, carry):
            for j in range(EDGE_CHUNK // 16):
                src_v[r, pl.ds(j * 16, 16)] = src_v[r, pl.ds(j * 16, 16)] + 1
            return carry

        lax.fori_loop(0, TILE_CHUNKS, bump, 0)
        zero_acc()
        plsc.subcore_barrier()
        edge_sweep()
        plsc.subcore_barrier()
        writeback(NC + cid)

    return k(table, src3, et3, dst3)


def _mlp_body(a00_ref, a01_ref, a10_ref, a11_ref, xl_ref, hb_ref,
              w1_ref, b1_ref, w2_ref, b2_ref, o_ref):
    xl = xl_ref[...]
    hb = hb_ref[...]
    w1 = w1_ref[...]
    s0 = a00_ref[...] + a01_ref[...] + xl[:, :HFEAT] + hb[:, :HFEAT]
    s1 = a10_ref[...] + a11_ref[...] + xl[:, HFEAT:] + hb[:, HFEAT:]
    t = (jnp.dot(s0, w1[:HFEAT, :], preferred_element_type=jnp.float32)
         + jnp.dot(s1, w1[HFEAT:, :], preferred_element_type=jnp.float32)
         + b1_ref[...])
    h1 = jnp.maximum(t, 0.0)
    o_ref[...] = jnp.maximum(
        jnp.dot(h1, w2_ref[...], preferred_element_type=jnp.float32)
        + b2_ref[...], 0.0)


def _mlp(aggp, xl, hb, W1, b1, W2, b2):
    nb = N_NODES // ROW_BLOCK

    def plane_spec(p):
        return pl.BlockSpec((ROW_BLOCK, HFEAT), lambda i, p=p: (i + p * nb, 0))

    row_spec = pl.BlockSpec((ROW_BLOCK, FEAT), lambda i: (i, 0))
    mat_spec = pl.BlockSpec((FEAT, FEAT), lambda i: (0, 0))
    vec_spec = pl.BlockSpec((1, FEAT), lambda i: (0, 0))
    return pl.pallas_call(
        _mlp_body,
        grid=(nb,),
        in_specs=[plane_spec(0), plane_spec(1), plane_spec(2), plane_spec(3),
                  row_spec, vec_spec, mat_spec, vec_spec, mat_spec, vec_spec],
        out_specs=row_spec,
        out_shape=jax.ShapeDtypeStruct((N_NODES, FEAT), jnp.float32),
    )(aggp, aggp, aggp, aggp, xl, hb, W1, b1, W2, b2)


def kernel(x, edge_index, etypes, weight, w_comp, loop_weight, h_bias,
           W1, b1, W2, b2):
    eshape = (NW, TILE_CHUNKS, EDGE_CHUNK)
    src3 = edge_index[0].astype(jnp.int32).reshape(eshape)
    dst3 = edge_index[1].astype(jnp.int32).reshape(eshape)
    et3 = etypes.astype(jnp.int32).reshape(eshape)
    hall, xl = _project(x, weight, w_comp, loop_weight)
    table = hall.reshape(N_NODES * NUM_RELS * 2, HFEAT)
    aggp = _edge_agg(table, src3, et3, dst3)
    return _mlp(aggp, xl, h_bias.reshape(1, FEAT), W1,
                b1.reshape(1, FEAT), W2, b2.reshape(1, FEAT))


# speedup vs baseline: 30.9712x; 1.2168x over previous
import functools

import jax
import jax.numpy as jnp
from jax import lax
from jax.experimental import pallas as pl
from jax.experimental.pallas import tpu as pltpu
from jax.experimental.pallas import tpu_sc as plsc

N_NODES = 10000
N_EDGES = 320000
FEAT = 128
HFEAT = FEAT // 2
NUM_RELS = 8
NUM_BASES = 4

NC = 2
NS = 16
NW = NC * NS

EDGE_CHUNK = 80
CHUNK_ROWS = N_EDGES // EDGE_CHUNK
TILE_CHUNKS = CHUNK_ROWS // NW
ROWS_MAIN = 624
ZROWS = 48

ROW_BLOCK = 1000


def _proj_body(x_ref, w_ref, wc_ref, lw_ref, hall_ref, xl_ref):
    xb = x_ref[...]
    for r in range(NUM_RELS):
        wr = wc_ref[r, 0] * w_ref[0]
        for b in range(1, NUM_BASES):
            wr = wr + wc_ref[r, b] * w_ref[b]
        hall_ref[:, r * FEAT:(r + 1) * FEAT] = jnp.dot(
            xb, wr, preferred_element_type=jnp.float32)
    xl_ref[...] = jnp.dot(xb, lw_ref[...], preferred_element_type=jnp.float32)


def _project(x, weight, w_comp, loop_weight):
    return pl.pallas_call(
        _proj_body,
        grid=(N_NODES // ROW_BLOCK,),
        in_specs=[
            pl.BlockSpec((ROW_BLOCK, FEAT), lambda i: (i, 0)),
            pl.BlockSpec((NUM_BASES, FEAT, FEAT), lambda i: (0, 0, 0)),
            pl.BlockSpec(memory_space=pltpu.SMEM),
            pl.BlockSpec((FEAT, FEAT), lambda i: (0, 0)),
        ],
        out_specs=[
            pl.BlockSpec((ROW_BLOCK, NUM_RELS * FEAT), lambda i: (i, 0)),
            pl.BlockSpec((ROW_BLOCK, FEAT), lambda i: (i, 0)),
        ],
        out_shape=[
            jax.ShapeDtypeStruct((N_NODES, NUM_RELS * FEAT), jnp.float32),
            jax.ShapeDtypeStruct((N_NODES, FEAT), jnp.float32),
        ],
    )(x, weight, w_comp, loop_weight)


def _edge_agg(table, src3, et3, dst3):
    mesh = plsc.VectorSubcoreMesh(core_axis_name="c", subcore_axis_name="s")

    @functools.partial(
        pl.kernel,
        mesh=mesh,
        compiler_params=pltpu.CompilerParams(use_tc_tiling_on_sc=False),
        out_type=jax.ShapeDtypeStruct((2 * NC * N_NODES, HFEAT), jnp.float32),
        scratch_types=[
            pltpu.VMEM((TILE_CHUNKS, EDGE_CHUNK), jnp.int32),
            pltpu.VMEM((TILE_CHUNKS, EDGE_CHUNK), jnp.int32),
            pltpu.VMEM((TILE_CHUNKS, EDGE_CHUNK), jnp.int32),
            [pltpu.VMEM((EDGE_CHUNK, HFEAT), jnp.float32)] * 5,
            pltpu.VMEM((ZROWS, HFEAT), jnp.float32),
            pltpu.VMEM_SHARED((N_NODES, HFEAT), jnp.float32),
            [pltpu.SemaphoreType.DMA] * 5,
            [pltpu.SemaphoreType.DMA] * 5,
        ],
    )
    def k(table_hbm, src_hbm, et_hbm, dst_hbm, out_hbm,
          src_v, et_v, dst_v, bufs, zbuf_v, acc_sh, gsems, ssems):
        cid = lax.axis_index("c")
        tid = lax.axis_index("s")
        wid = tid * NC + cid
        nbase = tid * ROWS_MAIN
        last = tid == NS - 1

        zv = jnp.zeros((16,), jnp.float32)

        def zrow(r, carry):
            for j in range(HFEAT // 16):
                zbuf_v[r, pl.ds(j * 16, 16)] = zv
            return carry

        lax.fori_loop(0, ZROWS, zrow, 0)

        def zero_acc():
            for z in range(ROWS_MAIN // ZROWS):
                pltpu.sync_copy(zbuf_v,
                                acc_sh.at[pl.ds(nbase + z * ZROWS, ZROWS)])

            @pl.when(last)
            def _():
                pltpu.sync_copy(zbuf_v.at[pl.ds(0, 16)],
                                acc_sh.at[pl.ds(N_NODES - 16, 16)])

        def writeback(plane):
            obase = plane * N_NODES + nbase
            pltpu.sync_copy(acc_sh.at[pl.ds(nbase, ROWS_MAIN)],
                            out_hbm.at[pl.ds(obase, ROWS_MAIN)])

            @pl.when(last)
            def _():
                pltpu.sync_copy(
                    acc_sh.at[pl.ds(N_NODES - 16, 16)],
                    out_hbm.at[pl.ds(plane * N_NODES + N_NODES - 16, 16)])

        zero_acc()

        pltpu.sync_copy(src_hbm.at[wid], src_v)
        pltpu.sync_copy(et_hbm.at[wid], et_v)
        pltpu.sync_copy(dst_hbm.at[wid], dst_v)

        def gfill(r, carry):
            for j in range(EDGE_CHUNK // 16):
                s = src_v[r, pl.ds(j * 16, 16)]
                e = et_v[r, pl.ds(j * 16, 16)]
                src_v[r, pl.ds(j * 16, 16)] = (s * NUM_RELS + e) * 2
            return carry

        lax.fori_loop(0, TILE_CHUNKS, gfill, 0)
        plsc.subcore_barrier()

        NB = 5

        def gather(c, b):
            pltpu.async_copy(table_hbm.at[src_v.at[c]], bufs[b], gsems[b])

        def gwait(c, b):
            pltpu.make_async_copy(
                table_hbm.at[src_v.at[c]], bufs[b], gsems[b]).wait()

        def scatter(c, b):
            pltpu.async_copy(bufs[b], acc_sh.at[dst_v.at[c]], ssems[b],
                             add=True)

        def swait(c, b):
            pltpu.make_async_copy(
                bufs[b], acc_sh.at[dst_v.at[c]], ssems[b]).wait()

        def step(c, b, do_swait, do_gather):
            gwait(c, b)
            scatter(c, b)
            if do_swait:
                swait(c - 2, (b + 3) % NB)
            if do_gather:
                gather(c + 3, (b + 3) % NB)

        def edge_sweep():
            for b in range(3):
                gather(b, b)
            for b in range(NB):
                step(b, b, do_swait=b >= 2, do_gather=True)

            def group(g, carry):
                c0 = g * NB
                for b in range(NB):
                    step(c0 + b, b, do_swait=True, do_gather=True)
                return carry

            lax.fori_loop(1, TILE_CHUNKS // NB - 1, group, 0)
            c0 = TILE_CHUNKS - NB
            for b in range(NB):
                step(c0 + b, b, do_swait=True,
                     do_gather=c0 + b + 3 < TILE_CHUNKS)
            swait(TILE_CHUNKS - 2, (TILE_CHUNKS - 2) % NB)
            swait(TILE_CHUNKS - 1, (TILE_CHUNKS - 1) % NB)

        edge_sweep()
        plsc.subcore_barrier()
        writeback(cid)

        def bump(r, carry):
            for j in range(EDGE_CHUNK // 16):
                src_v[r, pl.ds(j * 16, 16)] = src_v[r, pl.ds(j * 16, 16)] + 1
            return carry

        lax.fori_loop(0, TILE_CHUNKS, bump, 0)
        zero_acc()
        plsc.subcore_barrier()
        edge_sweep()
        plsc.subcore_barrier()
        writeback(NC + cid)

    return k(table, src3, et3, dst3)


def _mlp_body(a00_ref, a01_ref, a10_ref, a11_ref, xl_ref, hb_ref,
              w1_ref, b1_ref, w2_ref, b2_ref, o_ref):
    xl = xl_ref[...]
    hb = hb_ref[...]
    w1 = w1_ref[...]
    s0 = a00_ref[...] + a01_ref[...] + xl[:, :HFEAT] + hb[:, :HFEAT]
    s1 = a10_ref[...] + a11_ref[...] + xl[:, HFEAT:] + hb[:, HFEAT:]
    t = (jnp.dot(s0, w1[:HFEAT, :], preferred_element_type=jnp.float32)
         + jnp.dot(s1, w1[HFEAT:, :], preferred_element_type=jnp.float32)
         + b1_ref[...])
    h1 = jnp.maximum(t, 0.0)
    o_ref[...] = jnp.maximum(
        jnp.dot(h1, w2_ref[...], preferred_element_type=jnp.float32)
        + b2_ref[...], 0.0)


def _mlp(aggp, xl, hb, W1, b1, W2, b2):
    nb = N_NODES // ROW_BLOCK

    def plane_spec(p):
        return pl.BlockSpec((ROW_BLOCK, HFEAT), lambda i, p=p: (i + p * nb, 0))

    row_spec = pl.BlockSpec((ROW_BLOCK, FEAT), lambda i: (i, 0))
    mat_spec = pl.BlockSpec((FEAT, FEAT), lambda i: (0, 0))
    vec_spec = pl.BlockSpec((1, FEAT), lambda i: (0, 0))
    return pl.pallas_call(
        _mlp_body,
        grid=(nb,),
        in_specs=[plane_spec(0), plane_spec(1), plane_spec(2), plane_spec(3),
                  row_spec, vec_spec, mat_spec, vec_spec, mat_spec, vec_spec],
        out_specs=row_spec,
        out_shape=jax.ShapeDtypeStruct((N_NODES, FEAT), jnp.float32),
    )(aggp, aggp, aggp, aggp, xl, hb, W1, b1, W2, b2)


def kernel(x, edge_index, etypes, weight, w_comp, loop_weight, h_bias,
           W1, b1, W2, b2):
    eshape = (NW, TILE_CHUNKS, EDGE_CHUNK)
    src3 = edge_index[0].astype(jnp.int32).reshape(eshape)
    dst3 = edge_index[1].astype(jnp.int32).reshape(eshape)
    et3 = etypes.astype(jnp.int32).reshape(eshape)
    hall, xl = _project(x, weight, w_comp, loop_weight)
    table = hall.reshape(N_NODES * NUM_RELS * 2, HFEAT)
    aggp = _edge_agg(table, src3, et3, dst3)
    return _mlp(aggp, xl, h_bias.reshape(1, FEAT), W1,
                b1.reshape(1, FEAT), W2, b2.reshape(1, FEAT))


# speedup vs baseline: 34.0797x; 1.1004x over previous
import functools

import jax
import jax.numpy as jnp
from jax import lax
from jax.experimental import pallas as pl
from jax.experimental.pallas import tpu as pltpu
from jax.experimental.pallas import tpu_sc as plsc

N_NODES = 10000
N_EDGES = 320000
FEAT = 128
HFEAT = FEAT // 2
NUM_RELS = 8
NUM_BASES = 4

NC = 2
NS = 16
NW = NC * NS

EDGE_CHUNK = 80
CHUNK_ROWS = N_EDGES // EDGE_CHUNK
TILE_CHUNKS = CHUNK_ROWS // NW
ROWS_MAIN = 624
ZROWS = 48

ROW_BLOCK = 1000


def _proj_body(x_ref, w_ref, wc_ref, lw_ref, hall_ref, xl_ref):
    xb = x_ref[...]
    for r in range(NUM_RELS):
        wr = wc_ref[r, 0] * w_ref[0]
        for b in range(1, NUM_BASES):
            wr = wr + wc_ref[r, b] * w_ref[b]
        pr = jnp.dot(xb, wr, preferred_element_type=jnp.float32)
        hall_ref[:, r, :, :] = pr.reshape(ROW_BLOCK // 8, 8, FEAT)
    xl_ref[...] = jnp.dot(xb, lw_ref[...], preferred_element_type=jnp.float32)


def _project(x, weight, w_comp, loop_weight):
    return pl.pallas_call(
        _proj_body,
        grid=(N_NODES // ROW_BLOCK,),
        in_specs=[
            pl.BlockSpec((ROW_BLOCK, FEAT), lambda i: (i, 0)),
            pl.BlockSpec((NUM_BASES, FEAT, FEAT), lambda i: (0, 0, 0)),
            pl.BlockSpec(memory_space=pltpu.SMEM),
            pl.BlockSpec((FEAT, FEAT), lambda i: (0, 0)),
        ],
        out_specs=[
            pl.BlockSpec((ROW_BLOCK // 8, NUM_RELS, 8, FEAT),
                         lambda i: (i, 0, 0, 0)),
            pl.BlockSpec((ROW_BLOCK, FEAT), lambda i: (i, 0)),
        ],
        out_shape=[
            jax.ShapeDtypeStruct((N_NODES // 8, NUM_RELS, 8, FEAT),
                                 jnp.float32),
            jax.ShapeDtypeStruct((N_NODES, FEAT), jnp.float32),
        ],
    )(x, weight, w_comp, loop_weight)


def _edge_agg(table, src3, et3, dst3):
    mesh = plsc.VectorSubcoreMesh(core_axis_name="c", subcore_axis_name="s")

    @functools.partial(
        pl.kernel,
        mesh=mesh,
        compiler_params=pltpu.CompilerParams(use_tc_tiling_on_sc=False),
        out_type=jax.ShapeDtypeStruct((2 * NC * N_NODES, HFEAT), jnp.float32),
        scratch_types=[
            pltpu.VMEM((TILE_CHUNKS, EDGE_CHUNK), jnp.int32),
            pltpu.VMEM((TILE_CHUNKS, EDGE_CHUNK), jnp.int32),
            pltpu.VMEM((TILE_CHUNKS, EDGE_CHUNK), jnp.int32),
            [pltpu.VMEM((EDGE_CHUNK, HFEAT), jnp.float32)] * 5,
            pltpu.VMEM((ZROWS, HFEAT), jnp.float32),
            pltpu.VMEM_SHARED((N_NODES, HFEAT), jnp.float32),
            [pltpu.SemaphoreType.DMA] * 5,
            [pltpu.SemaphoreType.DMA] * 5,
        ],
    )
    def k(table_hbm, src_hbm, et_hbm, dst_hbm, out_hbm,
          src_v, et_v, dst_v, bufs, zbuf_v, acc_sh, gsems, ssems):
        cid = lax.axis_index("c")
        tid = lax.axis_index("s")
        wid = tid * NC + cid
        nbase = tid * ROWS_MAIN
        last = tid == NS - 1

        zv = jnp.zeros((16,), jnp.float32)

        def zrow(r, carry):
            for j in range(HFEAT // 16):
                zbuf_v[r, pl.ds(j * 16, 16)] = zv
            return carry

        lax.fori_loop(0, ZROWS, zrow, 0)

        def zero_acc():
            for z in range(ROWS_MAIN // ZROWS):
                pltpu.sync_copy(zbuf_v,
                                acc_sh.at[pl.ds(nbase + z * ZROWS, ZROWS)])

            @pl.when(last)
            def _():
                pltpu.sync_copy(zbuf_v.at[pl.ds(0, 16)],
                                acc_sh.at[pl.ds(N_NODES - 16, 16)])

        def writeback(plane):
            obase = plane * N_NODES + nbase
            pltpu.sync_copy(acc_sh.at[pl.ds(nbase, ROWS_MAIN)],
                            out_hbm.at[pl.ds(obase, ROWS_MAIN)])

            @pl.when(last)
            def _():
                pltpu.sync_copy(
                    acc_sh.at[pl.ds(N_NODES - 16, 16)],
                    out_hbm.at[pl.ds(plane * N_NODES + N_NODES - 16, 16)])

        zero_acc()

        pltpu.sync_copy(src_hbm.at[wid], src_v)
        pltpu.sync_copy(et_hbm.at[wid], et_v)
        pltpu.sync_copy(dst_hbm.at[wid], dst_v)

        def gfill(r, carry):
            for j in range(EDGE_CHUNK // 16):
                s = src_v[r, pl.ds(j * 16, 16)]
                e = et_v[r, pl.ds(j * 16, 16)]
                src_v[r, pl.ds(j * 16, 16)] = (
                    (s >> 3) * 128 + e * 16 + (s & 7) * 2)
            return carry

        lax.fori_loop(0, TILE_CHUNKS, gfill, 0)
        plsc.subcore_barrier()

        NB = 5

        def gather(c, b):
            pltpu.async_copy(table_hbm.at[src_v.at[c]], bufs[b], gsems[b])

        def gwait(c, b):
            pltpu.make_async_copy(
                table_hbm.at[src_v.at[c]], bufs[b], gsems[b]).wait()

        def scatter(c, b):
            pltpu.async_copy(bufs[b], acc_sh.at[dst_v.at[c]], ssems[b],
                             add=True)

        def swait(c, b):
            pltpu.make_async_copy(
                bufs[b], acc_sh.at[dst_v.at[c]], ssems[b]).wait()

        def step(c, b, do_swait, do_gather):
            gwait(c, b)
            scatter(c, b)
            if do_swait:
                swait(c - 2, (b + 3) % NB)
            if do_gather:
                gather(c + 3, (b + 3) % NB)

        def edge_sweep():
            for b in range(3):
                gather(b, b)
            for b in range(NB):
                step(b, b, do_swait=b >= 2, do_gather=True)

            def group(g, carry):
                c0 = g * NB
                for b in range(NB):
                    step(c0 + b, b, do_swait=True, do_gather=True)
                return carry

            lax.fori_loop(1, TILE_CHUNKS // NB - 1, group, 0)
            c0 = TILE_CHUNKS - NB
            for b in range(NB):
                step(c0 + b, b, do_swait=True,
                     do_gather=c0 + b + 3 < TILE_CHUNKS)
            swait(TILE_CHUNKS - 2, (TILE_CHUNKS - 2) % NB)
            swait(TILE_CHUNKS - 1, (TILE_CHUNKS - 1) % NB)

        edge_sweep()
        plsc.subcore_barrier()
        writeback(cid)

        def bump(r, carry):
            for j in range(EDGE_CHUNK // 16):
                src_v[r, pl.ds(j * 16, 16)] = src_v[r, pl.ds(j * 16, 16)] + 1
            return carry

        lax.fori_loop(0, TILE_CHUNKS, bump, 0)
        zero_acc()
        plsc.subcore_barrier()
        edge_sweep()
        plsc.subcore_barrier()
        writeback(NC + cid)

    return k(table, src3, et3, dst3)


def _mlp_body(a00_ref, a01_ref, a10_ref, a11_ref, xl_ref, hb_ref,
              w1_ref, b1_ref, w2_ref, b2_ref, o_ref):
    xl = xl_ref[...]
    hb = hb_ref[...]
    w1 = w1_ref[...]
    s0 = a00_ref[...] + a01_ref[...] + xl[:, :HFEAT] + hb[:, :HFEAT]
    s1 = a10_ref[...] + a11_ref[...] + xl[:, HFEAT:] + hb[:, HFEAT:]
    t = (jnp.dot(s0, w1[:HFEAT, :], preferred_element_type=jnp.float32)
         + jnp.dot(s1, w1[HFEAT:, :], preferred_element_type=jnp.float32)
         + b1_ref[...])
    h1 = jnp.maximum(t, 0.0)
    o_ref[...] = jnp.maximum(
        jnp.dot(h1, w2_ref[...], preferred_element_type=jnp.float32)
        + b2_ref[...], 0.0)


def _mlp(aggp, xl, hb, W1, b1, W2, b2):
    nb = N_NODES // ROW_BLOCK

    def plane_spec(p):
        return pl.BlockSpec((ROW_BLOCK, HFEAT), lambda i, p=p: (i + p * nb, 0))

    row_spec = pl.BlockSpec((ROW_BLOCK, FEAT), lambda i: (i, 0))
    mat_spec = pl.BlockSpec((FEAT, FEAT), lambda i: (0, 0))
    vec_spec = pl.BlockSpec((1, FEAT), lambda i: (0, 0))
    return pl.pallas_call(
        _mlp_body,
        grid=(nb,),
        in_specs=[plane_spec(0), plane_spec(1), plane_spec(2), plane_spec(3),
                  row_spec, vec_spec, mat_spec, vec_spec, mat_spec, vec_spec],
        out_specs=row_spec,
        out_shape=jax.ShapeDtypeStruct((N_NODES, FEAT), jnp.float32),
    )(aggp, aggp, aggp, aggp, xl, hb, W1, b1, W2, b2)


def kernel(x, edge_index, etypes, weight, w_comp, loop_weight, h_bias,
           W1, b1, W2, b2):
    eshape = (NW, TILE_CHUNKS, EDGE_CHUNK)
    src3 = edge_index[0].astype(jnp.int32).reshape(eshape)
    dst3 = edge_index[1].astype(jnp.int32).reshape(eshape)
    et3 = etypes.astype(jnp.int32).reshape(eshape)
    hall4, xl = _project(x, weight, w_comp, loop_weight)
    table = hall4.reshape(N_NODES * NUM_RELS * 2, HFEAT)
    aggp = _edge_agg(table, src3, et3, dst3)
    return _mlp(aggp, xl, h_bias.reshape(1, FEAT), W1,
                b1.reshape(1, FEAT), W2, b2.reshape(1, FEAT))


# speedup vs baseline: 40.1399x; 1.1778x over previous
import functools

import jax
import jax.numpy as jnp
from jax import lax
from jax.experimental import pallas as pl
from jax.experimental.pallas import tpu as pltpu
from jax.experimental.pallas import tpu_sc as plsc

N_NODES = 10000
N_EDGES = 320000
FEAT = 128
HFEAT = FEAT // 2
NUM_RELS = 8
NUM_BASES = 4

NC = 2
NS = 16
NW = NC * NS

EDGE_CHUNK = 80
CHUNK_ROWS = N_EDGES // EDGE_CHUNK
TILE_CHUNKS = CHUNK_ROWS // NW
ROWS_MAIN = 624
ZROWS = 48

ROW_BLOCK = 1000


def _proj_body(x_ref, w_ref, wc_ref, lw_ref, hall_ref, xl_ref):
    xb = x_ref[...]
    for r in range(NUM_RELS):
        wr = wc_ref[r, 0] * w_ref[0]
        for b in range(1, NUM_BASES):
            wr = wr + wc_ref[r, b] * w_ref[b]
        pr = jnp.dot(xb, wr, preferred_element_type=jnp.float32)
        hall_ref[:, r, :, :] = pr.reshape(ROW_BLOCK // 8, 8, FEAT)
    xl_ref[...] = jnp.dot(xb, lw_ref[...], preferred_element_type=jnp.float32)


def _project(x, weight, w_comp, loop_weight):
    return pl.pallas_call(
        _proj_body,
        grid=(N_NODES // ROW_BLOCK,),
        in_specs=[
            pl.BlockSpec((ROW_BLOCK, FEAT), lambda i: (i, 0)),
            pl.BlockSpec((NUM_BASES, FEAT, FEAT), lambda i: (0, 0, 0)),
            pl.BlockSpec(memory_space=pltpu.SMEM),
            pl.BlockSpec((FEAT, FEAT), lambda i: (0, 0)),
        ],
        out_specs=[
            pl.BlockSpec((ROW_BLOCK // 8, NUM_RELS, 8, FEAT),
                         lambda i: (i, 0, 0, 0)),
            pl.BlockSpec((ROW_BLOCK, FEAT), lambda i: (i, 0)),
        ],
        out_shape=[
            jax.ShapeDtypeStruct((N_NODES // 8, NUM_RELS, 8, FEAT),
                                 jnp.float32),
            jax.ShapeDtypeStruct((N_NODES, FEAT), jnp.float32),
        ],
    )(x, weight, w_comp, loop_weight)


def _edge_agg(table, edge_index, etypes):
    mesh = plsc.VectorSubcoreMesh(core_axis_name="c", subcore_axis_name="s")
    EPT = TILE_CHUNKS * EDGE_CHUNK

    @functools.partial(
        pl.kernel,
        mesh=mesh,
        compiler_params=pltpu.CompilerParams(use_tc_tiling_on_sc=False),
        out_type=jax.ShapeDtypeStruct((NC * N_NODES, FEAT), jnp.float32),
        scratch_types=[
            pltpu.VMEM((EPT,), jnp.int32),
            pltpu.VMEM((EPT,), jnp.int32),
            pltpu.VMEM((EPT,), jnp.int32),
            pltpu.VMEM((TILE_CHUNKS, EDGE_CHUNK), jnp.int32),
            pltpu.VMEM((TILE_CHUNKS, EDGE_CHUNK), jnp.int32),
            [pltpu.VMEM((EDGE_CHUNK, HFEAT), jnp.float32)] * 5,
            pltpu.VMEM((ZROWS, HFEAT), jnp.float32),
            pltpu.VMEM_SHARED((N_NODES, HFEAT), jnp.float32),
            [pltpu.SemaphoreType.DMA] * 5,
            [pltpu.SemaphoreType.DMA] * 5,
        ],
    )
    def k(table_hbm, ei_hbm, et_hbm, out_hbm,
          srcf, etf, dstf, src_v, dst_v, bufs, zbuf_v, acc_sh, gsems, ssems):
        cid = lax.axis_index("c")
        tid = lax.axis_index("s")
        wid = tid * NC + cid
        nbase = tid * ROWS_MAIN
        last = tid == NS - 1

        zv = jnp.zeros((16,), jnp.float32)

        def zrow(r, carry):
            for j in range(HFEAT // 16):
                zbuf_v[r, pl.ds(j * 16, 16)] = zv
            return carry

        lax.fori_loop(0, ZROWS, zrow, 0)

        def zero_acc():
            for z in range(ROWS_MAIN // ZROWS):
                pltpu.sync_copy(zbuf_v,
                                acc_sh.at[pl.ds(nbase + z * ZROWS, ZROWS)])

            @pl.when(last)
            def _():
                pltpu.sync_copy(zbuf_v.at[pl.ds(0, 16)],
                                acc_sh.at[pl.ds(N_NODES - 16, 16)])

        def writeback(p):
            obase = cid * N_NODES + nbase
            pltpu.sync_copy(
                acc_sh.at[pl.ds(nbase, ROWS_MAIN)],
                out_hbm.at[pl.ds(obase, ROWS_MAIN), pl.ds(p * HFEAT, HFEAT)])

            @pl.when(last)
            def _():
                pltpu.sync_copy(
                    acc_sh.at[pl.ds(N_NODES - 16, 16)],
                    out_hbm.at[pl.ds(cid * N_NODES + N_NODES - 16, 16),
                               pl.ds(p * HFEAT, HFEAT)])

        zero_acc()

        ebase = wid * EPT
        pltpu.sync_copy(ei_hbm.at[0, pl.ds(ebase, EPT)], srcf)
        pltpu.sync_copy(et_hbm.at[pl.ds(ebase, EPT)], etf)
        pltpu.sync_copy(ei_hbm.at[1, pl.ds(ebase, EPT)], dstf)

        def gfill(r, carry):
            for j in range(EDGE_CHUNK // 16):
                off = r * EDGE_CHUNK + j * 16
                s = srcf[pl.ds(off, 16)]
                e = etf[pl.ds(off, 16)]
                src_v[r, pl.ds(j * 16, 16)] = (
                    (s >> 3) * 128 + e * 16 + (s & 7) * 2)
                dst_v[r, pl.ds(j * 16, 16)] = dstf[pl.ds(off, 16)]
            return carry

        lax.fori_loop(0, TILE_CHUNKS, gfill, 0)
        plsc.subcore_barrier()

        NB = 5

        def gather(c, b):
            pltpu.async_copy(table_hbm.at[src_v.at[c]], bufs[b], gsems[b])

        def gwait(c, b):
            pltpu.make_async_copy(
                table_hbm.at[src_v.at[c]], bufs[b], gsems[b]).wait()

        def scatter(c, b):
            pltpu.async_copy(bufs[b], acc_sh.at[dst_v.at[c]], ssems[b],
                             add=True)

        def swait(c, b):
            pltpu.make_async_copy(
                bufs[b], acc_sh.at[dst_v.at[c]], ssems[b]).wait()

        def step(c, b, do_swait, do_gather):
            gwait(c, b)
            scatter(c, b)
            if do_swait:
                swait(c - 2, (b + 3) % NB)
            if do_gather:
                gather(c + 3, (b + 3) % NB)

        def edge_sweep():
            for b in range(3):
                gather(b, b)
            for b in range(NB):
                step(b, b, do_swait=b >= 2, do_gather=True)

            def group(g, carry):
                c0 = g * NB
                for b in range(NB):
                    step(c0 + b, b, do_swait=True, do_gather=True)
                return carry

            lax.fori_loop(1, TILE_CHUNKS // NB - 1, group, 0)
            c0 = TILE_CHUNKS - NB
            for b in range(NB):
                step(c0 + b, b, do_swait=True,
                     do_gather=c0 + b + 3 < TILE_CHUNKS)
            swait(TILE_CHUNKS - 2, (TILE_CHUNKS - 2) % NB)
            swait(TILE_CHUNKS - 1, (TILE_CHUNKS - 1) % NB)

        edge_sweep()
        plsc.subcore_barrier()
        writeback(0)

        def bump(r, carry):
            for j in range(EDGE_CHUNK // 16):
                src_v[r, pl.ds(j * 16, 16)] = src_v[r, pl.ds(j * 16, 16)] + 1
            return carry

        lax.fori_loop(0, TILE_CHUNKS, bump, 0)
        zero_acc()
        plsc.subcore_barrier()
        edge_sweep()
        plsc.subcore_barrier()
        writeback(1)

    return k(table, edge_index, etypes)


def _mlp_body(a0_ref, a1_ref, xl_ref, hb_ref, w1_ref, b1_ref, w2_ref,
              b2_ref, o_ref):
    s = a0_ref[...] + a1_ref[...] + xl_ref[...] + hb_ref[...]
    h1 = jnp.maximum(
        jnp.dot(s, w1_ref[...], preferred_element_type=jnp.float32)
        + b1_ref[...], 0.0)
    o_ref[...] = jnp.maximum(
        jnp.dot(h1, w2_ref[...], preferred_element_type=jnp.float32)
        + b2_ref[...], 0.0)


def _mlp(aggp, xl, hb, W1, b1, W2, b2):
    nb = N_NODES // ROW_BLOCK
    a0_spec = pl.BlockSpec((ROW_BLOCK, FEAT), lambda i: (i, 0))
    a1_spec = pl.BlockSpec((ROW_BLOCK, FEAT), lambda i: (i + nb, 0))
    row_spec = pl.BlockSpec((ROW_BLOCK, FEAT), lambda i: (i, 0))
    mat_spec = pl.BlockSpec((FEAT, FEAT), lambda i: (0, 0))
    vec_spec = pl.BlockSpec((1, FEAT), lambda i: (0, 0))
    return pl.pallas_call(
        _mlp_body,
        grid=(nb,),
        in_specs=[a0_spec, a1_spec, row_spec, vec_spec, mat_spec,
                  vec_spec, mat_spec, vec_spec],
        out_specs=row_spec,
        out_shape=jax.ShapeDtypeStruct((N_NODES, FEAT), jnp.float32),
    )(aggp, aggp, xl, hb, W1, b1, W2, b2)


def kernel(x, edge_index, etypes, weight, w_comp, loop_weight, h_bias,
           W1, b1, W2, b2):
    hall4, xl = _project(x, weight, w_comp, loop_weight)
    table = hall4.reshape(N_NODES * NUM_RELS * 2, HFEAT)
    aggp = _edge_agg(table, edge_index.astype(jnp.int32),
                     etypes.astype(jnp.int32))
    return _mlp(aggp, xl, h_bias.reshape(1, FEAT), W1,
                b1.reshape(1, FEAT), W2, b2.reshape(1, FEAT))


# speedup vs baseline: 42.3450x; 1.0549x over previous
import functools

import jax
import jax.numpy as jnp
from jax import lax
from jax.experimental import pallas as pl
from jax.experimental.pallas import tpu as pltpu
from jax.experimental.pallas import tpu_sc as plsc

N_NODES = 10000
N_EDGES = 320000
FEAT = 128
HFEAT = FEAT // 2
NUM_RELS = 8
NUM_BASES = 4

NC = 2
NS = 16
NW = NC * NS

EDGE_CHUNK = 80
CHUNK_ROWS = N_EDGES // EDGE_CHUNK
TILE_CHUNKS = CHUNK_ROWS // NW
ROWS_MAIN = 624
ZROWS = 48

ROW_BLOCK = 1000


def _proj_body(x_ref, w_ref, wc_ref, lw_ref, hall_ref, xl_ref):
    xb = x_ref[...]
    for r in range(NUM_RELS):
        wr = wc_ref[r, 0] * w_ref[0]
        for b in range(1, NUM_BASES):
            wr = wr + wc_ref[r, b] * w_ref[b]
        pr = jnp.dot(xb, wr, preferred_element_type=jnp.float32)
        hall_ref[:, r, :, :] = pr.reshape(ROW_BLOCK // 8, 8, FEAT)
    xl_ref[...] = jnp.dot(xb, lw_ref[...], preferred_element_type=jnp.float32)


def _project(x, weight, w_comp, loop_weight):
    return pl.pallas_call(
        _proj_body,
        grid=(N_NODES // ROW_BLOCK,),
        in_specs=[
            pl.BlockSpec((ROW_BLOCK, FEAT), lambda i: (i, 0)),
            pl.BlockSpec((NUM_BASES, FEAT, FEAT), lambda i: (0, 0, 0)),
            pl.BlockSpec(memory_space=pltpu.SMEM),
            pl.BlockSpec((FEAT, FEAT), lambda i: (0, 0)),
        ],
        out_specs=[
            pl.BlockSpec((ROW_BLOCK // 8, NUM_RELS, 8, FEAT),
                         lambda i: (i, 0, 0, 0)),
            pl.BlockSpec((ROW_BLOCK, FEAT), lambda i: (i, 0)),
        ],
        out_shape=[
            jax.ShapeDtypeStruct((N_NODES // 8, NUM_RELS, 8, FEAT),
                                 jnp.float32),
            jax.ShapeDtypeStruct((N_NODES, FEAT), jnp.float32),
        ],
    )(x, weight, w_comp, loop_weight)


def _edge_agg(table, edge_index, etypes):
    mesh = plsc.VectorSubcoreMesh(core_axis_name="c", subcore_axis_name="s")
    EPT = TILE_CHUNKS * EDGE_CHUNK

    @functools.partial(
        pl.kernel,
        mesh=mesh,
        compiler_params=pltpu.CompilerParams(use_tc_tiling_on_sc=False),
        out_type=jax.ShapeDtypeStruct((NC * N_NODES, FEAT), jnp.float32),
        scratch_types=[
            pltpu.VMEM((EPT,), jnp.int32),
            pltpu.VMEM((EPT,), jnp.int32),
            pltpu.VMEM((EPT,), jnp.int32),
            pltpu.VMEM((TILE_CHUNKS, EDGE_CHUNK), jnp.int32),
            pltpu.VMEM((TILE_CHUNKS, EDGE_CHUNK), jnp.int32),
            [pltpu.VMEM((EDGE_CHUNK, HFEAT), jnp.float32)] * 5,
            pltpu.VMEM((ZROWS, HFEAT), jnp.float32),
            pltpu.VMEM_SHARED((N_NODES, HFEAT), jnp.float32),
            [pltpu.SemaphoreType.DMA] * 5,
            [pltpu.SemaphoreType.DMA] * 5,
        ],
    )
    def k(table_hbm, ei_hbm, et_hbm, out_hbm,
          srcf, etf, dstf, src_v, dst_v, bufs, zbuf_v, acc_sh, gsems, ssems):
        cid = lax.axis_index("c")
        tid = lax.axis_index("s")
        wid = tid * NC + cid
        nbase = tid * ROWS_MAIN
        last = tid == NS - 1

        zv = jnp.zeros((16,), jnp.float32)

        def zrow(r, carry):
            for j in range(HFEAT // 16):
                zbuf_v[r, pl.ds(j * 16, 16)] = zv
            return carry

        lax.fori_loop(0, ZROWS, zrow, 0)

        def zero_acc():
            for z in range(ROWS_MAIN // ZROWS):
                pltpu.sync_copy(zbuf_v,
                                acc_sh.at[pl.ds(nbase + z * ZROWS, ZROWS)])

            @pl.when(last)
            def _():
                pltpu.sync_copy(zbuf_v.at[pl.ds(0, 16)],
                                acc_sh.at[pl.ds(N_NODES - 16, 16)])

        def writeback(p):
            obase = cid * N_NODES + nbase
            pltpu.sync_copy(
                acc_sh.at[pl.ds(nbase, ROWS_MAIN)],
                out_hbm.at[pl.ds(obase, ROWS_MAIN), pl.ds(p * HFEAT, HFEAT)])

            @pl.when(last)
            def _():
                pltpu.sync_copy(
                    acc_sh.at[pl.ds(N_NODES - 16, 16)],
                    out_hbm.at[pl.ds(cid * N_NODES + N_NODES - 16, 16),
                               pl.ds(p * HFEAT, HFEAT)])

        zero_acc()

        ebase = wid * EPT
        pltpu.sync_copy(ei_hbm.at[0, pl.ds(ebase, EPT)], srcf)
        pltpu.sync_copy(et_hbm.at[pl.ds(ebase, EPT)], etf)
        pltpu.sync_copy(ei_hbm.at[1, pl.ds(ebase, EPT)], dstf)

        def gfill(r, carry):
            for j in range(EDGE_CHUNK // 16):
                off = r * EDGE_CHUNK + j * 16
                s = srcf[pl.ds(off, 16)]
                e = etf[pl.ds(off, 16)]
                src_v[r, pl.ds(j * 16, 16)] = (
                    (s >> 3) * 128 + e * 16 + (s & 7) * 2)
                dst_v[r, pl.ds(j * 16, 16)] = dstf[pl.ds(off, 16)]
            return carry

        lax.fori_loop(0, TILE_CHUNKS, gfill, 0)
        plsc.subcore_barrier()

        NB = 5

        def gather(c, b):
            pltpu.async_copy(table_hbm.at[src_v.at[c]], bufs[b], gsems[b])

        def gwait(c, b):
            pltpu.make_async_copy(
                table_hbm.at[src_v.at[c]], bufs[b], gsems[b]).wait()

        def scatter(c, b):
            pltpu.async_copy(bufs[b], acc_sh.at[dst_v.at[c]], ssems[b],
                             add=True)

        def swait(c, b):
            pltpu.make_async_copy(
                bufs[b], acc_sh.at[dst_v.at[c]], ssems[b]).wait()

        LEAD = 4

        def step(c, b, do_swait, do_gather):
            gwait(c, b)
            scatter(c, b)
            if do_swait:
                swait(c - (NB - LEAD), (b + LEAD) % NB)
            if do_gather:
                gather(c + LEAD, (b + LEAD) % NB)

        def edge_sweep():
            for b in range(LEAD):
                gather(b, b)
            for b in range(NB):
                step(b, b, do_swait=b >= NB - LEAD, do_gather=True)

            def group(g, carry):
                c0 = g * NB
                for b in range(NB):
                    step(c0 + b, b, do_swait=True, do_gather=True)
                return carry

            lax.fori_loop(1, TILE_CHUNKS // NB - 1, group, 0)
            c0 = TILE_CHUNKS - NB
            for b in range(NB):
                step(c0 + b, b, do_swait=True,
                     do_gather=c0 + b + LEAD < TILE_CHUNKS)
            for c in range(TILE_CHUNKS - (NB - LEAD), TILE_CHUNKS):
                swait(c, c % NB)

        edge_sweep()
        plsc.subcore_barrier()
        writeback(0)

        def bump(r, carry):
            for j in range(EDGE_CHUNK // 16):
                src_v[r, pl.ds(j * 16, 16)] = src_v[r, pl.ds(j * 16, 16)] + 1
            return carry

        lax.fori_loop(0, TILE_CHUNKS, bump, 0)
        zero_acc()
        plsc.subcore_barrier()
        edge_sweep()
        plsc.subcore_barrier()
        writeback(1)

    return k(table, edge_index, etypes)


def _mlp_body(a0_ref, a1_ref, xl_ref, hb_ref, w1_ref, b1_ref, w2_ref,
              b2_ref, o_ref):
    s = a0_ref[...] + a1_ref[...] + xl_ref[...] + hb_ref[...]
    h1 = jnp.maximum(
        jnp.dot(s, w1_ref[...], preferred_element_type=jnp.float32)
        + b1_ref[...], 0.0)
    o_ref[...] = jnp.maximum(
        jnp.dot(h1, w2_ref[...], preferred_element_type=jnp.float32)
        + b2_ref[...], 0.0)


def _mlp(aggp, xl, hb, W1, b1, W2, b2):
    nb = N_NODES // ROW_BLOCK
    a0_spec = pl.BlockSpec((ROW_BLOCK, FEAT), lambda i: (i, 0))
    a1_spec = pl.BlockSpec((ROW_BLOCK, FEAT), lambda i: (i + nb, 0))
    row_spec = pl.BlockSpec((ROW_BLOCK, FEAT), lambda i: (i, 0))
    mat_spec = pl.BlockSpec((FEAT, FEAT), lambda i: (0, 0))
    vec_spec = pl.BlockSpec((1, FEAT), lambda i: (0, 0))
    return pl.pallas_call(
        _mlp_body,
        grid=(nb,),
        in_specs=[a0_spec, a1_spec, row_spec, vec_spec, mat_spec,
                  vec_spec, mat_spec, vec_spec],
        out_specs=row_spec,
        out_shape=jax.ShapeDtypeStruct((N_NODES, FEAT), jnp.float32),
    )(aggp, aggp, xl, hb, W1, b1, W2, b2)


def kernel(x, edge_index, etypes, weight, w_comp, loop_weight, h_bias,
           W1, b1, W2, b2):
    hall4, xl = _project(x, weight, w_comp, loop_weight)
    table = hall4.reshape(N_NODES * NUM_RELS * 2, HFEAT)
    aggp = _edge_agg(table, edge_index.astype(jnp.int32),
                     etypes.astype(jnp.int32))
    return _mlp(aggp, xl, h_bias.reshape(1, FEAT), W1,
                b1.reshape(1, FEAT), W2, b2.reshape(1, FEAT))


# speedup vs baseline: 43.1654x; 1.0194x over previous
import functools

import jax
import jax.numpy as jnp
from jax import lax
from jax.experimental import pallas as pl
from jax.experimental.pallas import tpu as pltpu
from jax.experimental.pallas import tpu_sc as plsc

N_NODES = 10000
N_EDGES = 320000
FEAT = 128
HFEAT = FEAT // 2
NUM_RELS = 8
NUM_BASES = 4

NC = 2
NS = 16
NW = NC * NS

EDGE_CHUNK = 80
CHUNK_ROWS = N_EDGES // EDGE_CHUNK
TILE_CHUNKS = CHUNK_ROWS // NW
ROWS_MAIN = 624
ZROWS = 48

ROW_BLOCK = 1000


def _proj_body(x_ref, w_ref, wc_ref, lw_ref, hall_ref, xl_ref):
    xb = x_ref[...]
    for r in range(NUM_RELS):
        wr = wc_ref[r, 0] * w_ref[0]
        for b in range(1, NUM_BASES):
            wr = wr + wc_ref[r, b] * w_ref[b]
        pr = jnp.dot(xb, wr, preferred_element_type=jnp.float32)
        hall_ref[:, r, :, :] = pr.reshape(ROW_BLOCK // 8, 8, FEAT)
    xl_ref[...] = jnp.dot(xb, lw_ref[...], preferred_element_type=jnp.float32)


def _project(x, weight, w_comp, loop_weight):
    return pl.pallas_call(
        _proj_body,
        grid=(N_NODES // ROW_BLOCK,),
        in_specs=[
            pl.BlockSpec((ROW_BLOCK, FEAT), lambda i: (i, 0)),
            pl.BlockSpec((NUM_BASES, FEAT, FEAT), lambda i: (0, 0, 0)),
            pl.BlockSpec(memory_space=pltpu.SMEM),
            pl.BlockSpec((FEAT, FEAT), lambda i: (0, 0)),
        ],
        out_specs=[
            pl.BlockSpec((ROW_BLOCK // 8, NUM_RELS, 8, FEAT),
                         lambda i: (i, 0, 0, 0)),
            pl.BlockSpec((ROW_BLOCK, FEAT), lambda i: (i, 0)),
        ],
        out_shape=[
            jax.ShapeDtypeStruct((N_NODES // 8, NUM_RELS, 8, FEAT),
                                 jnp.float32),
            jax.ShapeDtypeStruct((N_NODES, FEAT), jnp.float32),
        ],
    )(x, weight, w_comp, loop_weight)


def _edge_agg(table, ei3, et3):
    mesh = plsc.VectorSubcoreMesh(core_axis_name="c", subcore_axis_name="s")

    @functools.partial(
        pl.kernel,
        mesh=mesh,
        compiler_params=pltpu.CompilerParams(use_tc_tiling_on_sc=False),
        out_type=jax.ShapeDtypeStruct((NC * N_NODES, FEAT), jnp.float32),
        scratch_types=[
            pltpu.VMEM((TILE_CHUNKS, EDGE_CHUNK), jnp.int32),
            pltpu.VMEM((TILE_CHUNKS, EDGE_CHUNK), jnp.int32),
            pltpu.VMEM((TILE_CHUNKS, EDGE_CHUNK), jnp.int32),
            [pltpu.VMEM((EDGE_CHUNK, HFEAT), jnp.float32)] * 10,
            pltpu.VMEM((ZROWS, HFEAT), jnp.float32),
            pltpu.VMEM_SHARED((N_NODES, HFEAT), jnp.float32),
            [pltpu.SemaphoreType.DMA] * 10,
            [pltpu.SemaphoreType.DMA] * 10,
        ],
    )
    def k(table_hbm, ei_hbm, et_hbm, out_hbm,
          src_v, et_v, dst_v, bufs, zbuf_v, acc_sh, gsems, ssems):
        cid = lax.axis_index("c")
        tid = lax.axis_index("s")
        wid = tid * NC + cid
        nbase = tid * ROWS_MAIN
        last = tid == NS - 1

        zv = jnp.zeros((16,), jnp.float32)

        def zrow(r, carry):
            for j in range(HFEAT // 16):
                zbuf_v[r, pl.ds(j * 16, 16)] = zv
            return carry

        lax.fori_loop(0, ZROWS, zrow, 0)

        def zero_acc():
            for z in range(ROWS_MAIN // ZROWS):
                pltpu.sync_copy(zbuf_v,
                                acc_sh.at[pl.ds(nbase + z * ZROWS, ZROWS)])

            @pl.when(last)
            def _():
                pltpu.sync_copy(zbuf_v.at[pl.ds(0, 16)],
                                acc_sh.at[pl.ds(N_NODES - 16, 16)])

        def writeback(p):
            obase = cid * N_NODES + nbase
            pltpu.sync_copy(
                acc_sh.at[pl.ds(nbase, ROWS_MAIN)],
                out_hbm.at[pl.ds(obase, ROWS_MAIN), pl.ds(p * HFEAT, HFEAT)])

            @pl.when(last)
            def _():
                pltpu.sync_copy(
                    acc_sh.at[pl.ds(N_NODES - 16, 16)],
                    out_hbm.at[pl.ds(cid * N_NODES + N_NODES - 16, 16),
                               pl.ds(p * HFEAT, HFEAT)])

        zero_acc()

        rbase = wid * TILE_CHUNKS
        pltpu.sync_copy(ei_hbm.at[0, pl.ds(rbase, TILE_CHUNKS)], src_v)
        pltpu.sync_copy(et_hbm.at[pl.ds(rbase, TILE_CHUNKS)], et_v)
        pltpu.sync_copy(ei_hbm.at[1, pl.ds(rbase, TILE_CHUNKS)], dst_v)

        def gfill(r, carry):
            for j in range(EDGE_CHUNK // 16):
                s = src_v[r, pl.ds(j * 16, 16)]
                e = et_v[r, pl.ds(j * 16, 16)]
                src_v[r, pl.ds(j * 16, 16)] = (
                    (s >> 3) * 128 + e * 16 + (s & 7) * 2)
            return carry

        lax.fori_loop(0, TILE_CHUNKS, gfill, 0)
        plsc.subcore_barrier()

        NB = 10

        def gather(c, b):
            pltpu.async_copy(table_hbm.at[src_v.at[c]], bufs[b], gsems[b])

        def gwait(c, b):
            pltpu.make_async_copy(
                table_hbm.at[src_v.at[c]], bufs[b], gsems[b]).wait()

        def scatter(c, b):
            pltpu.async_copy(bufs[b], acc_sh.at[dst_v.at[c]], ssems[b],
                             add=True)

        def swait(c, b):
            pltpu.make_async_copy(
                bufs[b], acc_sh.at[dst_v.at[c]], ssems[b]).wait()

        LEAD = 8
        NFULL = TILE_CHUNKS // NB
        NTAIL = TILE_CHUNKS % NB

        def step(c, b, do_swait, do_gather):
            gwait(c, b)
            scatter(c, b)
            if do_swait:
                swait(c - (NB - LEAD), (b + LEAD) % NB)
            if do_gather:
                gather(c + LEAD, (b + LEAD) % NB)

        def edge_sweep():
            for b in range(LEAD):
                gather(b, b)
            for b in range(NB):
                step(b, b, do_swait=b >= NB - LEAD, do_gather=True)

            def group(g, carry):
                c0 = g * NB
                for b in range(NB):
                    step(c0 + b, b, do_swait=True, do_gather=True)
                return carry

            lax.fori_loop(1, NFULL - 1, group, 0)
            c0 = (NFULL - 1) * NB
            for b in range(NB):
                step(c0 + b, b, do_swait=True,
                     do_gather=c0 + b + LEAD < TILE_CHUNKS)
            c0 = NFULL * NB
            for t in range(NTAIL):
                step(c0 + t, (c0 + t) % NB, do_swait=True, do_gather=False)
            for c in range(TILE_CHUNKS - (NB - LEAD), TILE_CHUNKS):
                swait(c, c % NB)

        edge_sweep()
        plsc.subcore_barrier()
        writeback(0)

        def bump(r, carry):
            for j in range(EDGE_CHUNK // 16):
                src_v[r, pl.ds(j * 16, 16)] = src_v[r, pl.ds(j * 16, 16)] + 1
            return carry

        lax.fori_loop(0, TILE_CHUNKS, bump, 0)
        zero_acc()
        plsc.subcore_barrier()
        edge_sweep()
        plsc.subcore_barrier()
        writeback(1)

    return k(table, ei3, et3)


def _mlp_body(a0_ref, a1_ref, xl_ref, hb_ref, w1_ref, b1_ref, w2_ref,
              b2_ref, o_ref):
    s = a0_ref[...] + a1_ref[...] + xl_ref[...] + hb_ref[...]
    h1 = jnp.maximum(
        jnp.dot(s, w1_ref[...], preferred_element_type=jnp.float32)
        + b1_ref[...], 0.0)
    o_ref[...] = jnp.maximum(
        jnp.dot(h1, w2_ref[...], preferred_element_type=jnp.float32)
        + b2_ref[...], 0.0)


def _mlp(aggp, xl, hb, W1, b1, W2, b2):
    nb = N_NODES // ROW_BLOCK
    a0_spec = pl.BlockSpec((ROW_BLOCK, FEAT), lambda i: (i, 0))
    a1_spec = pl.BlockSpec((ROW_BLOCK, FEAT), lambda i: (i + nb, 0))
    row_spec = pl.BlockSpec((ROW_BLOCK, FEAT), lambda i: (i, 0))
    mat_spec = pl.BlockSpec((FEAT, FEAT), lambda i: (0, 0))
    vec_spec = pl.BlockSpec((1, FEAT), lambda i: (0, 0))
    return pl.pallas_call(
        _mlp_body,
        grid=(nb,),
        in_specs=[a0_spec, a1_spec, row_spec, vec_spec, mat_spec,
                  vec_spec, mat_spec, vec_spec],
        out_specs=row_spec,
        out_shape=jax.ShapeDtypeStruct((N_NODES, FEAT), jnp.float32),
    )(aggp, aggp, xl, hb, W1, b1, W2, b2)


def kernel(x, edge_index, etypes, weight, w_comp, loop_weight, h_bias,
           W1, b1, W2, b2):
    hall4, xl = _project(x, weight, w_comp, loop_weight)
    table = hall4.reshape(N_NODES * NUM_RELS * 2, HFEAT)
    ei3 = edge_index.astype(jnp.int32).reshape(2, CHUNK_ROWS, EDGE_CHUNK)
    et3 = etypes.astype(jnp.int32).reshape(CHUNK_ROWS, EDGE_CHUNK)
    aggp = _edge_agg(table, ei3, et3)
    return _mlp(aggp, xl, h_bias.reshape(1, FEAT), W1,
                b1.reshape(1, FEAT), W2, b2.reshape(1, FEAT))


# speedup vs baseline: 43.5957x; 1.0100x over previous
import functools

import jax
import jax.numpy as jnp
from jax import lax
from jax.experimental import pallas as pl
from jax.experimental.pallas import tpu as pltpu
from jax.experimental.pallas import tpu_sc as plsc

N_NODES = 10000
N_EDGES = 320000
FEAT = 128
HFEAT = FEAT // 2
NUM_RELS = 8
NUM_BASES = 4

NC = 2
NS = 16
NW = NC * NS

EDGE_CHUNK = 80
CHUNK_ROWS = N_EDGES // EDGE_CHUNK
TILE_CHUNKS = CHUNK_ROWS // NW
ROWS_MAIN = 624
ZROWS = 48

ROW_BLOCK = 1000


def _proj_body(x_ref, w_ref, wc_ref, lw_ref, hall_ref, xl_ref):
    xb = x_ref[...]
    for r in range(NUM_RELS):
        wr = wc_ref[r, 0] * w_ref[0]
        for b in range(1, NUM_BASES):
            wr = wr + wc_ref[r, b] * w_ref[b]
        pr = jnp.dot(xb, wr, preferred_element_type=jnp.float32)
        hall_ref[:, r, :, :] = pr.reshape(ROW_BLOCK // 8, 8, FEAT)
    xl_ref[...] = jnp.dot(xb, lw_ref[...], preferred_element_type=jnp.float32)


def _project(x, weight, w_comp, loop_weight):
    return pl.pallas_call(
        _proj_body,
        grid=(N_NODES // ROW_BLOCK,),
        in_specs=[
            pl.BlockSpec((ROW_BLOCK, FEAT), lambda i: (i, 0)),
            pl.BlockSpec((NUM_BASES, FEAT, FEAT), lambda i: (0, 0, 0)),
            pl.BlockSpec(memory_space=pltpu.SMEM),
            pl.BlockSpec((FEAT, FEAT), lambda i: (0, 0)),
        ],
        out_specs=[
            pl.BlockSpec((ROW_BLOCK // 8, NUM_RELS, 8, FEAT),
                         lambda i: (i, 0, 0, 0)),
            pl.BlockSpec((ROW_BLOCK, FEAT), lambda i: (i, 0)),
        ],
        out_shape=[
            jax.ShapeDtypeStruct((N_NODES // 8, NUM_RELS, 8, FEAT),
                                 jnp.float32),
            jax.ShapeDtypeStruct((N_NODES, FEAT), jnp.float32),
        ],
    )(x, weight, w_comp, loop_weight)


def _edge_agg(table, ei3, et3, xl):
    mesh = plsc.VectorSubcoreMesh(core_axis_name="c", subcore_axis_name="s")

    @functools.partial(
        pl.kernel,
        mesh=mesh,
        compiler_params=pltpu.CompilerParams(use_tc_tiling_on_sc=False),
        out_type=jax.ShapeDtypeStruct((NC * N_NODES, FEAT), jnp.float32),
        scratch_types=[
            pltpu.VMEM((TILE_CHUNKS, EDGE_CHUNK), jnp.int32),
            pltpu.VMEM((TILE_CHUNKS, EDGE_CHUNK), jnp.int32),
            pltpu.VMEM((TILE_CHUNKS, EDGE_CHUNK), jnp.int32),
            [pltpu.VMEM((EDGE_CHUNK, HFEAT), jnp.float32)] * 10,
            pltpu.VMEM((ZROWS, HFEAT), jnp.float32),
            pltpu.VMEM_SHARED((N_NODES, HFEAT), jnp.float32),
            [pltpu.SemaphoreType.DMA] * 10,
            [pltpu.SemaphoreType.DMA] * 10,
        ],
    )
    def k(table_hbm, ei_hbm, et_hbm, xl_hbm, out_hbm,
          src_v, et_v, dst_v, bufs, zbuf_v, acc_sh, gsems, ssems):
        cid = lax.axis_index("c")
        tid = lax.axis_index("s")
        wid = tid * NC + cid
        nbase = tid * ROWS_MAIN
        last = tid == NS - 1

        zv = jnp.zeros((16,), jnp.float32)

        def zrow(r, carry):
            for j in range(HFEAT // 16):
                zbuf_v[r, pl.ds(j * 16, 16)] = zv
            return carry

        lax.fori_loop(0, ZROWS, zrow, 0)

        def zero_acc(p):
            @pl.when(cid == 0)
            def _():
                pltpu.sync_copy(
                    xl_hbm.at[pl.ds(nbase, ROWS_MAIN),
                              pl.ds(p * HFEAT, HFEAT)],
                    acc_sh.at[pl.ds(nbase, ROWS_MAIN)])

                @pl.when(last)
                def _():
                    pltpu.sync_copy(
                        xl_hbm.at[pl.ds(N_NODES - 16, 16),
                                  pl.ds(p * HFEAT, HFEAT)],
                        acc_sh.at[pl.ds(N_NODES - 16, 16)])

            @pl.when(cid != 0)
            def _():
                for z in range(ROWS_MAIN // ZROWS):
                    pltpu.sync_copy(zbuf_v,
                                    acc_sh.at[pl.ds(nbase + z * ZROWS, ZROWS)])

                @pl.when(last)
                def _():
                    pltpu.sync_copy(zbuf_v.at[pl.ds(0, 16)],
                                    acc_sh.at[pl.ds(N_NODES - 16, 16)])

        def writeback(p):
            obase = cid * N_NODES + nbase
            pltpu.sync_copy(
                acc_sh.at[pl.ds(nbase, ROWS_MAIN)],
                out_hbm.at[pl.ds(obase, ROWS_MAIN), pl.ds(p * HFEAT, HFEAT)])

            @pl.when(last)
            def _():
                pltpu.sync_copy(
                    acc_sh.at[pl.ds(N_NODES - 16, 16)],
                    out_hbm.at[pl.ds(cid * N_NODES + N_NODES - 16, 16),
                               pl.ds(p * HFEAT, HFEAT)])

        zero_acc(0)

        rbase = wid * TILE_CHUNKS
        pltpu.sync_copy(ei_hbm.at[0, pl.ds(rbase, TILE_CHUNKS)], src_v)
        pltpu.sync_copy(et_hbm.at[pl.ds(rbase, TILE_CHUNKS)], et_v)
        pltpu.sync_copy(ei_hbm.at[1, pl.ds(rbase, TILE_CHUNKS)], dst_v)

        def gfill(r, carry):
            for j in range(EDGE_CHUNK // 16):
                s = src_v[r, pl.ds(j * 16, 16)]
                e = et_v[r, pl.ds(j * 16, 16)]
                src_v[r, pl.ds(j * 16, 16)] = (
                    (s >> 3) * 128 + e * 16 + (s & 7) * 2)
            return carry

        lax.fori_loop(0, TILE_CHUNKS, gfill, 0)
        plsc.subcore_barrier()

        NB = 10

        def gather(c, b):
            pltpu.async_copy(table_hbm.at[src_v.at[c]], bufs[b], gsems[b])

        def gwait(c, b):
            pltpu.make_async_copy(
                table_hbm.at[src_v.at[c]], bufs[b], gsems[b]).wait()

        def scatter(c, b):
            pltpu.async_copy(bufs[b], acc_sh.at[dst_v.at[c]], ssems[b],
                             add=True)

        def swait(c, b):
            pltpu.make_async_copy(
                bufs[b], acc_sh.at[dst_v.at[c]], ssems[b]).wait()

        LEAD = 8
        NFULL = TILE_CHUNKS // NB
        NTAIL = TILE_CHUNKS % NB

        def step(c, b, do_swait, do_gather):
            gwait(c, b)
            scatter(c, b)
            if do_swait:
                swait(c - (NB - LEAD), (b + LEAD) % NB)
            if do_gather:
                gather(c + LEAD, (b + LEAD) % NB)

        def edge_sweep():
            for b in range(LEAD):
                gather(b, b)
            for b in range(NB):
                step(b, b, do_swait=b >= NB - LEAD, do_gather=True)

            def group(g, carry):
                c0 = g * NB
                for b in range(NB):
                    step(c0 + b, b, do_swait=True, do_gather=True)
                return carry

            lax.fori_loop(1, NFULL - 1, group, 0)
            c0 = (NFULL - 1) * NB
            for b in range(NB):
                step(c0 + b, b, do_swait=True,
                     do_gather=c0 + b + LEAD < TILE_CHUNKS)
            c0 = NFULL * NB
            for t in range(NTAIL):
                step(c0 + t, (c0 + t) % NB, do_swait=True, do_gather=False)
            for c in range(TILE_CHUNKS - (NB - LEAD), TILE_CHUNKS):
                swait(c, c % NB)

        edge_sweep()
        plsc.subcore_barrier()
        writeback(0)

        def bump(r, carry):
            for j in range(EDGE_CHUNK // 16):
                src_v[r, pl.ds(j * 16, 16)] = src_v[r, pl.ds(j * 16, 16)] + 1
            return carry

        lax.fori_loop(0, TILE_CHUNKS, bump, 0)
        zero_acc(1)
        plsc.subcore_barrier()
        edge_sweep()
        plsc.subcore_barrier()
        writeback(1)

    return k(table, ei3, et3, xl)


def _mlp_body(a0_ref, a1_ref, hb_ref, w1_ref, b1_ref, w2_ref,
              b2_ref, o_ref):
    s = a0_ref[...] + a1_ref[...] + hb_ref[...]
    h1 = jnp.maximum(
        jnp.dot(s, w1_ref[...], preferred_element_type=jnp.float32)
        + b1_ref[...], 0.0)
    o_ref[...] = jnp.maximum(
        jnp.dot(h1, w2_ref[...], preferred_element_type=jnp.float32)
        + b2_ref[...], 0.0)


def _mlp(aggp, hb, W1, b1, W2, b2):
    nb = N_NODES // ROW_BLOCK
    a0_spec = pl.BlockSpec((ROW_BLOCK, FEAT), lambda i: (i, 0))
    a1_spec = pl.BlockSpec((ROW_BLOCK, FEAT), lambda i: (i + nb, 0))
    row_spec = pl.BlockSpec((ROW_BLOCK, FEAT), lambda i: (i, 0))
    mat_spec = pl.BlockSpec((FEAT, FEAT), lambda i: (0, 0))
    vec_spec = pl.BlockSpec((1, FEAT), lambda i: (0, 0))
    return pl.pallas_call(
        _mlp_body,
        grid=(nb,),
        in_specs=[a0_spec, a1_spec, vec_spec, mat_spec,
                  vec_spec, mat_spec, vec_spec],
        out_specs=row_spec,
        out_shape=jax.ShapeDtypeStruct((N_NODES, FEAT), jnp.float32),
    )(aggp, aggp, hb, W1, b1, W2, b2)


def kernel(x, edge_index, etypes, weight, w_comp, loop_weight, h_bias,
           W1, b1, W2, b2):
    hall4, xl = _project(x, weight, w_comp, loop_weight)
    table = hall4.reshape(N_NODES * NUM_RELS * 2, HFEAT)
    ei3 = edge_index.astype(jnp.int32).reshape(2, CHUNK_ROWS, EDGE_CHUNK)
    et3 = etypes.astype(jnp.int32).reshape(CHUNK_ROWS, EDGE_CHUNK)
    aggp = _edge_agg(table, ei3, et3, xl)
    return _mlp(aggp, h_bias.reshape(1, FEAT), W1,
                b1.reshape(1, FEAT), W2, b2.reshape(1, FEAT))
